# baseline (speedup 1.0000x reference)
import functools

import jax
import jax.numpy as jnp
from jax import lax
from jax.experimental import pallas as pl
from jax.experimental.pallas import tpu as pltpu
from jax.experimental.pallas import tpu_sc as plsc

D = 32
H = 50
RC = 16
RCP = RC + 1
NBUF = 2


@functools.lru_cache(maxsize=None)
def _make_gather(BS, V):
    info = plsc.get_sparse_core_info()
    NC, NS = info.num_cores, info.num_subcores
    NW = NC * NS
    rows_per_w = BS // NW
    nchunk = rows_per_w // RC
    assert rows_per_w % RC == 0

    mesh = plsc.VectorSubcoreMesh(core_axis_name="c", subcore_axis_name="s")

    @functools.partial(
        pl.kernel,
        mesh=mesh,
        out_type=jax.ShapeDtypeStruct((H, D, BS), jnp.float32),
        scratch_types=[
            *[pltpu.VMEM((RC, H), jnp.int32) for _ in range(NBUF)],
            *[pltpu.VMEM((RC * H, D), jnp.float32) for _ in range(NBUF)],
            *[pltpu.VMEM((H, D, RCP), jnp.float32) for _ in range(NBUF)],
            *[pltpu.SemaphoreType.DMA for _ in range(3 * NBUF)],
        ],
        compiler_params=pltpu.CompilerParams(use_tc_tiling_on_sc=False,
                                             needs_layout_passes=False),
    )
    def gather_kernel(ids_hbm, table_hbm, out_hbm, *scratch):
        idx = scratch[:NBUF]
        rows = scratch[NBUF:2 * NBUF]
        tbuf = scratch[2 * NBUF:3 * NBUF]
        sem_i = scratch[3 * NBUF:4 * NBUF]
        sem_g = scratch[4 * NBUF:5 * NBUF]
        sem_w = scratch[5 * NBUF:]
        wid = lax.axis_index("s") * NC + lax.axis_index("c")
        base = wid * rows_per_w

        def start_idx(g, b):
            pltpu.async_copy(
                ids_hbm.at[pl.ds(pl.multiple_of(base + g * RC, 8), RC), :],
                idx[b], sem_i[b])

        def wait_idx(b):
            pltpu.make_async_copy(ids_hbm.at[pl.ds(0, RC), :], idx[b],
                                  sem_i[b]).wait()

        def start_gathers(b):
            for r in range(RC):
                pltpu.async_copy(table_hbm.at[idx[b].at[r]],
                                 rows[b].at[pl.ds(r * H, H)], sem_g[b])

        def wait_gathers(b):
            pltpu.make_async_copy(table_hbm.at[pl.ds(0, RC * H)], rows[b],
                                  sem_g[b]).wait()

        iota = lax.iota(jnp.int32, RC)
        d_vecs = [d0 + iota for d0 in range(0, D, RC)]
        r_vecs = [jnp.full((RC,), r, jnp.int32) for r in range(RC)]

        def transpose(b):
            @plsc.parallel_loop(0, H, unroll=4)
            def hbody(h):
                h_vec = jnp.full((RC,), 0, jnp.int32) + h
                for r in range(RC):
                    for k in range(D // RC):
                        v = rows[b][r * H + h, pl.ds(k * RC, RC)]
                        plsc.store_scatter(
                            tbuf[b], [h_vec, d_vecs[k], r_vecs[r]], v)

        def start_writeback(g, b):
            pltpu.make_async_copy(
                tbuf[b].at[:, :, pl.ds(0, RC)],
                out_hbm.at[:, :, pl.ds(base + g * RC, RC)],
                sem_w[b]).start()

        def wait_writeback(b):
            pltpu.make_async_copy(
                tbuf[b].at[:, :, pl.ds(0, RC)],
                out_hbm.at[:, :, pl.ds(0, RC)], sem_w[b]).wait()

        start_idx(0, 0)

        def body(s, carry):
            for b in range(NBUF):
                g = s * NBUF + b
                bn = (b + 1) % NBUF

                @pl.when(g < nchunk)
                def _():
                    wait_idx(b)
                    start_gathers(b)

                @pl.when(g + 1 < nchunk)
                def _():
                    start_idx(g + 1, bn)

                @pl.when((g >= 1) & (g <= nchunk))
                def _():
                    wait_gathers(bn)

                    @pl.when(g >= 1 + NBUF)
                    def _():
                        wait_writeback(bn)

                    transpose(bn)
                    start_writeback(g - 1, bn)
            return carry

        lax.fori_loop(0, (nchunk + 2) // NBUF, body, 0)

        for b in range(NBUF):
            wait_writeback(b)

    return gather_kernel


def kernel(ids, table):
    bsz, hist = ids.shape
    out3 = _make_gather(bsz, table.shape[0])(ids, table)
    return jnp.transpose(out3, (2, 0, 1))


# speedup vs baseline: 1.0402x; 1.0402x over previous
import functools

import jax
import jax.numpy as jnp
from jax import lax
from jax.experimental import pallas as pl
from jax.experimental.pallas import tpu as pltpu
from jax.experimental.pallas import tpu_sc as plsc

D = 32
H = 50
RC = 16
RCP = RC + 1
NBUF = 2


@functools.lru_cache(maxsize=None)
def _make_gather(BS, V):
    info = plsc.get_sparse_core_info()
    NC, NS = info.num_cores, info.num_subcores
    NW = NC * NS
    rows_per_w = BS // NW
    nchunk = rows_per_w // RC
    assert rows_per_w % RC == 0

    mesh = plsc.VectorSubcoreMesh(core_axis_name="c", subcore_axis_name="s")

    @functools.partial(
        pl.kernel,
        mesh=mesh,
        out_type=jax.ShapeDtypeStruct((H, D, BS), jnp.float32),
        scratch_types=[
            *[pltpu.VMEM((RC, H), jnp.int32) for _ in range(NBUF)],
            *[pltpu.VMEM((RC * H, D), jnp.float32) for _ in range(NBUF)],
            *[pltpu.VMEM((H, D, RCP), jnp.float32) for _ in range(NBUF)],
            *[pltpu.SemaphoreType.DMA for _ in range(3 * NBUF)],
        ],
        compiler_params=pltpu.CompilerParams(use_tc_tiling_on_sc=False,
                                             needs_layout_passes=False),
    )
    def gather_kernel(ids_hbm, table_hbm, out_hbm, *scratch):
        idx = scratch[:NBUF]
        rows = scratch[NBUF:2 * NBUF]
        tbuf = scratch[2 * NBUF:3 * NBUF]
        sem_i = scratch[3 * NBUF:4 * NBUF]
        sem_g = scratch[4 * NBUF:5 * NBUF]
        sem_w = scratch[5 * NBUF:]
        wid = lax.axis_index("s") * NC + lax.axis_index("c")
        base = wid * rows_per_w

        def start_idx(g, b):
            pltpu.async_copy(
                ids_hbm.at[pl.ds(pl.multiple_of(base + g * RC, 8), RC), :],
                idx[b], sem_i[b])

        def wait_idx(b):
            pltpu.make_async_copy(ids_hbm.at[pl.ds(0, RC), :], idx[b],
                                  sem_i[b]).wait()

        def start_gathers(b):
            for r in range(RC):
                pltpu.async_copy(table_hbm.at[idx[b].at[r]],
                                 rows[b].at[pl.ds(r * H, H)], sem_g[b])

        def wait_gathers(b):
            pltpu.make_async_copy(table_hbm.at[pl.ds(0, RC * H)], rows[b],
                                  sem_g[b]).wait()

        iota = lax.iota(jnp.int32, RC)
        d_vecs = [d0 + iota for d0 in range(0, D, RC)]
        r_vecs = [jnp.full((RC,), r, jnp.int32) for r in range(RC)]

        def transpose(b):
            @plsc.parallel_loop(0, H, unroll=2)
            def hbody(h):
                h_vec = jnp.full((RC,), 0, jnp.int32) + h
                for r in range(RC):
                    for k in range(D // RC):
                        v = rows[b][r * H + h, pl.ds(k * RC, RC)]
                        plsc.store_scatter(
                            tbuf[b], [h_vec, d_vecs[k], r_vecs[r]], v)

        def start_writeback(g, b):
            pltpu.make_async_copy(
                tbuf[b].at[:, :, pl.ds(0, RC)],
                out_hbm.at[:, :, pl.ds(base + g * RC, RC)],
                sem_w[b]).start()

        def wait_writeback(b):
            pltpu.make_async_copy(
                tbuf[b].at[:, :, pl.ds(0, RC)],
                out_hbm.at[:, :, pl.ds(0, RC)], sem_w[b]).wait()

        start_idx(0, 0)

        def body(s, carry):
            for b in range(NBUF):
                g = s * NBUF + b
                bn = (b + 1) % NBUF

                @pl.when(g < nchunk)
                def _():
                    wait_idx(b)
                    start_gathers(b)

                @pl.when(g + 1 < nchunk)
                def _():
                    start_idx(g + 1, bn)

                @pl.when((g >= 1) & (g <= nchunk))
                def _():
                    wait_gathers(bn)

                    @pl.when(g >= 1 + NBUF)
                    def _():
                        wait_writeback(bn)

                    transpose(bn)
                    start_writeback(g - 1, bn)
            return carry

        lax.fori_loop(0, (nchunk + 2) // NBUF, body, 0)

        for b in range(NBUF):
            wait_writeback(b)

    return gather_kernel


def kernel(ids, table):
    bsz, hist = ids.shape
    out3 = _make_gather(bsz, table.shape[0])(ids, table)
    return jnp.transpose(out3, (2, 0, 1))
